# baseline (speedup 1.0000x reference)
import functools

import jax
import jax.numpy as jnp
from jax import lax
from jax.experimental import pallas as pl
from jax.experimental.pallas import tpu as pltpu
from jax.experimental.pallas import tpu_sc as plsc

_NC = 2
_NS = 16
_NW = _NC * _NS
_LANES = 16
_SUBL = 200
_SUBW = 224

_PROJ_BLK = 8192


def _project_body(table_ref, w_ref, out0_ref, out1_ref):
    r = lax.dot_general(
        w_ref[...], table_ref[...],
        dimension_numbers=(((1,), (0,)), ((), ())),
        preferred_element_type=jnp.float32)
    out0_ref[...] = r[0]
    out1_ref[...] = r[1]


def _project(table_t, W):
    D, V = table_t.shape
    C = W.shape[0]
    return pl.pallas_call(
        _project_body,
        grid=(pl.cdiv(V, _PROJ_BLK),),
        in_specs=[
            pl.BlockSpec((D, _PROJ_BLK), lambda i: (0, i)),
            pl.BlockSpec((C, D), lambda i: (0, 0)),
        ],
        out_specs=[
            pl.BlockSpec((_PROJ_BLK,), lambda i: (i,)),
            pl.BlockSpec((_PROJ_BLK,), lambda i: (i,)),
        ],
        out_shape=[
            jax.ShapeDtypeStruct((V,), jnp.float32),
            jax.ShapeDtypeStruct((V,), jnp.float32),
        ],
    )(table_t, W)


def _sc_gather(p0, p1, idx4, L, bpw):
    lpad = _SUBW
    nsl = _SUBW // _LANES

    mesh = plsc.VectorSubcoreMesh(core_axis_name="c", subcore_axis_name="s")

    @functools.partial(
        pl.kernel,
        mesh=mesh,
        out_type=jax.ShapeDtypeStruct((_NW, 2, lpad), jnp.float32),
        scratch_types=[
            pltpu.VMEM((bpw // 2, 1, _SUBW), jnp.int32),
            pltpu.VMEM((_SUBW,), jnp.float32),
            pltpu.VMEM((_SUBW,), jnp.float32),
            pltpu.VMEM((_SUBW,), jnp.float32),
            pltpu.VMEM((_SUBW,), jnp.float32),
            pltpu.VMEM((lpad,), jnp.float32),
            pltpu.VMEM((lpad,), jnp.float32),
            pltpu.VMEM((2 * lpad,), jnp.float32),
            pltpu.SemaphoreType.DMA,
            pltpu.SemaphoreType.DMA,
        ],
    )
    def sc_kernel(p0_hbm, p1_hbm, idx_hbm, out_hbm, idx_vm,
                  b00, b01, b10, b11, acc0, acc1, dummy, sem0, sem1):
        wid = lax.axis_index("s") * _NC + lax.axis_index("c")
        hb = bpw // 2

        bufs = ((b00, b01), (b10, b11))
        accs = (acc0, acc1)
        zf = jnp.zeros((_LANES,), jnp.float32)
        for a in accs:
            for k in range(nsl):
                a[pl.ds(k * _LANES, _LANES)] = zf

        def fire(b, par, sem):
            pltpu.async_copy(p0_hbm.at[idx_vm.at[b, 0]], bufs[par][0], sem)
            pltpu.async_copy(p1_hbm.at[idx_vm.at[b, 0]], bufs[par][1], sem)

        def drain(par, sem):
            pltpu.make_async_copy(
                p0_hbm.at[pl.ds(0, 2 * lpad)], dummy, sem).wait()

        def accumulate(par):
            for k in range(nsl):
                sl = pl.ds(k * _LANES, _LANES)
                for c in range(2):
                    accs[c][sl] = accs[c][sl] + bufs[par][c][sl]

        def body(i, carry):
            b0 = 2 * i
            fire(b0 + 1, 1, sem1)
            drain(0, sem0)
            accumulate(0)

            @pl.when(b0 + 2 < hb)
            def _():
                fire(b0 + 2, 0, sem0)

            drain(1, sem1)
            accumulate(1)
            return carry

        for h in range(2):
            pltpu.sync_copy(idx_hbm.at[wid, pl.ds(h * hb, hb)], idx_vm)
            fire(0, 0, sem0)
            lax.fori_loop(0, hb // 2, body, 0)

        pltpu.sync_copy(acc0, out_hbm.at[wid, 0])
        pltpu.sync_copy(acc1, out_hbm.at[wid, 1])

    return sc_kernel(p0, p1, idx4)


def _finish_body(part_ref, b_ref, out_ref, *, inv_b, L):
    s = jnp.sum(part_ref[...], axis=0)
    lpad = s.shape[1]
    li = lax.broadcasted_iota(jnp.int32, (L, lpad), 0)
    ji = lax.broadcasted_iota(jnp.int32, (L, lpad), 1)
    sel = (li == ji).astype(jnp.float32)
    o = lax.dot_general(
        sel, s, dimension_numbers=(((1,), (1,)), ((), ())),
        preferred_element_type=jnp.float32)
    out_ref[...] = o * inv_b + b_ref[...]


def _finish(partials, b2d, B, L):
    return pl.pallas_call(
        functools.partial(_finish_body, inv_b=1.0 / B, L=L),
        out_shape=jax.ShapeDtypeStruct((L, 2), jnp.float32),
    )(partials, b2d)


def kernel(input_sentence_indices, table, W, b):
    idx = input_sentence_indices.astype(jnp.int32)
    B, L = idx.shape
    V, D = table.shape
    C = W.shape[0]
    assert C == 2 and B % _NW == 0 and L == _SUBL

    bpw = B // _NW
    p0, p1 = _project(table.T, W)
    idx4 = jnp.pad(
        idx.reshape(_NW, bpw, 1, _SUBL),
        ((0, 0), (0, 0), (0, 0), (0, _SUBW - _SUBL)))
    partials = _sc_gather(p0, p1, idx4, L, bpw)
    return _finish(partials, b.reshape(1, C), B, L)


# speedup vs baseline: 4.6575x; 4.6575x over previous
import functools

import jax
import jax.numpy as jnp
from jax import lax
from jax.experimental import pallas as pl
from jax.experimental.pallas import tpu as pltpu
from jax.experimental.pallas import tpu_sc as plsc

_NC = 2
_NS = 16
_NW = _NC * _NS
_CH = 128
_LANES = 16

_PROJ_BLK = 8192


def _project_body(table_ref, w_ref, out0_ref, out1_ref):
    r = lax.dot_general(
        w_ref[...], table_ref[...],
        dimension_numbers=(((1,), (0,)), ((), ())),
        preferred_element_type=jnp.float32)
    out0_ref[...] = r[0]
    out1_ref[...] = r[1]


def _project(table_t, W):
    D, V = table_t.shape
    C = W.shape[0]
    return pl.pallas_call(
        _project_body,
        grid=(pl.cdiv(V, _PROJ_BLK),),
        in_specs=[
            pl.BlockSpec((D, _PROJ_BLK), lambda i: (0, i)),
            pl.BlockSpec((C, D), lambda i: (0, 0)),
        ],
        out_specs=[
            pl.BlockSpec((_PROJ_BLK,), lambda i: (i,)),
            pl.BlockSpec((_PROJ_BLK,), lambda i: (i,)),
        ],
        out_shape=[
            jax.ShapeDtypeStruct((V,), jnp.float32),
            jax.ShapeDtypeStruct((V,), jnp.float32),
        ],
    )(table_t, W)


def _sc_gather(p0, p1, idx3, L, bpw):
    lpad = ((L + _LANES - 1) // _LANES) * _LANES
    nsl = lpad // _LANES
    ch2 = L - _CH

    mesh = plsc.VectorSubcoreMesh(core_axis_name="c", subcore_axis_name="s")

    @functools.partial(
        pl.kernel,
        mesh=mesh,
        out_type=jax.ShapeDtypeStruct((_NW, 2, lpad), jnp.float32),
        scratch_types=[
            pltpu.VMEM((bpw // 2, L), jnp.int32),
            pltpu.VMEM((lpad,), jnp.float32),
            pltpu.VMEM((lpad,), jnp.float32),
            pltpu.VMEM((lpad,), jnp.float32),
            pltpu.VMEM((lpad,), jnp.float32),
            pltpu.VMEM((lpad,), jnp.float32),
            pltpu.VMEM((lpad,), jnp.float32),
            pltpu.SemaphoreType.DMA,
            pltpu.SemaphoreType.DMA,
        ],
    )
    def sc_kernel(p0_hbm, p1_hbm, idx_hbm, out_hbm, idx_vm,
                  b00, b01, b10, b11, acc0, acc1, sem0, sem1):
        wid = lax.axis_index("s") * _NC + lax.axis_index("c")
        hb = bpw // 2

        bufs = ((b00, b01), (b10, b11))
        accs = (acc0, acc1)
        zf = jnp.zeros((_LANES,), jnp.float32)
        for a in accs:
            for k in range(nsl):
                a[pl.ds(k * _LANES, _LANES)] = zf
        for pair in bufs:
            for bb in pair:
                bb[pl.ds(lpad - _LANES, _LANES)] = zf

        def fire(b, par, sem):
            for c, pt in ((0, p0_hbm), (1, p1_hbm)):
                pltpu.async_copy(
                    pt.at[idx_vm.at[b, pl.ds(0, _CH)]],
                    bufs[par][c].at[pl.ds(0, _CH)], sem)
                pltpu.async_copy(
                    pt.at[idx_vm.at[b, pl.ds(_CH, ch2)]],
                    bufs[par][c].at[pl.ds(_CH, ch2)], sem)

        def drain(par, sem):
            for c in range(2):
                pltpu.make_async_copy(
                    p0_hbm.at[pl.ds(0, L)],
                    bufs[par][c].at[pl.ds(0, L)], sem).wait()

        def accumulate(par):
            for k in range(nsl):
                sl = pl.ds(k * _LANES, _LANES)
                for c in range(2):
                    accs[c][sl] = accs[c][sl] + bufs[par][c][sl]

        def body(i, carry):
            b0 = 2 * i
            fire(b0 + 1, 1, sem1)
            drain(0, sem0)
            accumulate(0)

            @pl.when(b0 + 2 < hb)
            def _():
                fire(b0 + 2, 0, sem0)

            drain(1, sem1)
            accumulate(1)
            return carry

        for h in range(2):
            pltpu.sync_copy(idx_hbm.at[wid, pl.ds(h * hb, hb)], idx_vm)
            fire(0, 0, sem0)
            lax.fori_loop(0, hb // 2, body, 0)

        pltpu.sync_copy(acc0, out_hbm.at[wid, 0])
        pltpu.sync_copy(acc1, out_hbm.at[wid, 1])

    return sc_kernel(p0, p1, idx3)


def _finish_body(part_ref, b_ref, out_ref, *, inv_b, L):
    s = jnp.sum(part_ref[...], axis=0)
    lpad = s.shape[1]
    li = lax.broadcasted_iota(jnp.int32, (L, lpad), 0)
    ji = lax.broadcasted_iota(jnp.int32, (L, lpad), 1)
    sel = (li == ji).astype(jnp.float32)
    o = lax.dot_general(
        sel, s, dimension_numbers=(((1,), (1,)), ((), ())),
        preferred_element_type=jnp.float32)
    out_ref[...] = o * inv_b + b_ref[...]


def _finish(partials, b2d, B, L):
    return pl.pallas_call(
        functools.partial(_finish_body, inv_b=1.0 / B, L=L),
        out_shape=jax.ShapeDtypeStruct((L, 2), jnp.float32),
    )(partials, b2d)


def kernel(input_sentence_indices, table, W, b):
    idx = input_sentence_indices.astype(jnp.int32)
    B, L = idx.shape
    V, D = table.shape
    C = W.shape[0]
    assert C == 2 and B % _NW == 0 and _CH < L <= 2 * _CH

    bpw = B // _NW
    p0, p1 = _project(table.T, W)
    idx3 = idx.reshape(_NW, bpw, L)
    partials = _sc_gather(p0, p1, idx3, L, bpw)
    return _finish(partials, b.reshape(1, C), B, L)


# speedup vs baseline: 5.2349x; 1.1240x over previous
import functools

import jax
import jax.numpy as jnp
from jax import lax
from jax.experimental import pallas as pl
from jax.experimental.pallas import tpu as pltpu
from jax.experimental.pallas import tpu_sc as plsc

_NC = 2
_NS = 16
_NW = _NC * _NS
_CH = 128
_LANES = 16
_GR = 16

_PROJ_BLK = 8192


def _project_body(table_ref, w_ref, out0_ref, out1_ref):
    r = lax.dot_general(
        w_ref[...], table_ref[...],
        dimension_numbers=(((1,), (0,)), ((), ())),
        preferred_element_type=jnp.float32)
    out0_ref[...] = r[0]
    out1_ref[...] = r[1]


def _project(table_t, W):
    D, V = table_t.shape
    C = W.shape[0]
    return pl.pallas_call(
        _project_body,
        grid=(pl.cdiv(V, _PROJ_BLK),),
        in_specs=[
            pl.BlockSpec((D, _PROJ_BLK), lambda i: (0, i)),
            pl.BlockSpec((C, D), lambda i: (0, 0)),
        ],
        out_specs=[
            pl.BlockSpec((_PROJ_BLK,), lambda i: (i,)),
            pl.BlockSpec((_PROJ_BLK,), lambda i: (i,)),
        ],
        out_shape=[
            jax.ShapeDtypeStruct((V,), jnp.float32),
            jax.ShapeDtypeStruct((V,), jnp.float32),
        ],
    )(table_t, W)


def _sc_gather(p0, p1, idx2, L, bpw):
    lpad = ((L + _LANES - 1) // _LANES) * _LANES
    nsl = lpad // _LANES
    gw = _GR * L
    nch = gw // _CH
    assert gw % _CH == 0

    mesh = plsc.VectorSubcoreMesh(core_axis_name="c", subcore_axis_name="s")

    @functools.partial(
        pl.kernel,
        mesh=mesh,
        out_type=jax.ShapeDtypeStruct((_NW, 2, lpad), jnp.float32),
        scratch_types=[
            pltpu.VMEM((bpw * L // 2,), jnp.int32),
            pltpu.VMEM((gw + 8,), jnp.float32),
            pltpu.VMEM((gw + 8,), jnp.float32),
            pltpu.VMEM((gw + 8,), jnp.float32),
            pltpu.VMEM((gw + 8,), jnp.float32),
            pltpu.VMEM((lpad,), jnp.float32),
            pltpu.VMEM((lpad,), jnp.float32),
            pltpu.SemaphoreType.DMA,
            pltpu.SemaphoreType.DMA,
        ],
    )
    def sc_kernel(p0_hbm, p1_hbm, idx_hbm, out_hbm, idx_vm,
                  b00, b01, b10, b11, acc0, acc1, sem0, sem1):
        wid = lax.axis_index("s") * _NC + lax.axis_index("c")
        hw = bpw * L // 2
        ng = hw // gw

        bufs = ((b00, b01), (b10, b11))
        accs = (acc0, acc1)
        zf = jnp.zeros((_LANES,), jnp.float32)
        for a in accs:
            for k in range(nsl):
                a[pl.ds(k * _LANES, _LANES)] = zf
        for pair in bufs:
            for bb in pair:
                bb[pl.ds(gw - 8, _LANES)] = zf

        def fire(g, par, sem):
            for c, pt in ((0, p0_hbm), (1, p1_hbm)):
                for k in range(nch):
                    sl = pl.ds(g * gw + k * _CH, _CH)
                    pltpu.async_copy(
                        pt.at[idx_vm.at[sl]],
                        bufs[par][c].at[pl.ds(k * _CH, _CH)], sem)

        def drain(par, sem):
            for c in range(2):
                pltpu.make_async_copy(
                    p0_hbm.at[pl.ds(0, gw)],
                    bufs[par][c].at[pl.ds(0, gw)], sem).wait()

        def accumulate(par):
            for r in range(_GR):
                for k in range(nsl):
                    a_sl = pl.ds(k * _LANES, _LANES)
                    b_sl = pl.ds(r * L + k * _LANES, _LANES)
                    for c in range(2):
                        accs[c][a_sl] = accs[c][a_sl] + bufs[par][c][b_sl]

        def body(i, carry):
            g0 = 2 * i
            fire(g0 + 1, 1, sem1)
            drain(0, sem0)
            accumulate(0)

            @pl.when(g0 + 2 < ng)
            def _():
                fire(g0 + 2, 0, sem0)

            drain(1, sem1)
            accumulate(1)
            return carry

        for h in range(2):
            pltpu.sync_copy(idx_hbm.at[wid, pl.ds(h * hw, hw)], idx_vm)
            fire(0, 0, sem0)
            lax.fori_loop(0, ng // 2, body, 0)

        pltpu.sync_copy(acc0, out_hbm.at[wid, 0])
        pltpu.sync_copy(acc1, out_hbm.at[wid, 1])

    return sc_kernel(p0, p1, idx2)


def _finish_body(part_ref, b_ref, out_ref, *, inv_b, L):
    s = jnp.sum(part_ref[...], axis=0)
    lpad = s.shape[1]
    li = lax.broadcasted_iota(jnp.int32, (L, lpad), 0)
    ji = lax.broadcasted_iota(jnp.int32, (L, lpad), 1)
    sel = (li == ji).astype(jnp.float32)
    o = lax.dot_general(
        sel, s, dimension_numbers=(((1,), (1,)), ((), ())),
        preferred_element_type=jnp.float32)
    out_ref[...] = o * inv_b + b_ref[...]


def _finish(partials, b2d, B, L):
    return pl.pallas_call(
        functools.partial(_finish_body, inv_b=1.0 / B, L=L),
        out_shape=jax.ShapeDtypeStruct((L, 2), jnp.float32),
    )(partials, b2d)


def kernel(input_sentence_indices, table, W, b):
    idx = input_sentence_indices.astype(jnp.int32)
    B, L = idx.shape
    V, D = table.shape
    C = W.shape[0]
    assert C == 2 and B % _NW == 0 and _CH < L <= 2 * _CH

    bpw = B // _NW
    p0, p1 = _project(table.T, W)
    idx2 = idx.reshape(_NW, bpw * L)
    partials = _sc_gather(p0, p1, idx2, L, bpw)
    return _finish(partials, b.reshape(1, C), B, L)


# speedup vs baseline: 5.8223x; 1.1122x over previous
import functools

import jax
import jax.numpy as jnp
from jax import lax
from jax.experimental import pallas as pl
from jax.experimental.pallas import tpu as pltpu
from jax.experimental.pallas import tpu_sc as plsc

_NC = 2
_NS = 16
_NW = _NC * _NS
_CH = 128
_LANES = 16
_GR = 16

_PROJ_BLK = 32768


def _project_body(table_ref, w_ref, out0_ref, out1_ref):
    r = lax.dot_general(
        w_ref[...], table_ref[...],
        dimension_numbers=(((1,), (0,)), ((), ())),
        preferred_element_type=jnp.float32)
    out0_ref[...] = r[0]
    out1_ref[...] = r[1]


def _project(table_t, W):
    D, V = table_t.shape
    C = W.shape[0]
    return pl.pallas_call(
        _project_body,
        grid=(pl.cdiv(V, _PROJ_BLK),),
        in_specs=[
            pl.BlockSpec((D, _PROJ_BLK), lambda i: (0, i)),
            pl.BlockSpec((C, D), lambda i: (0, 0)),
        ],
        out_specs=[
            pl.BlockSpec((_PROJ_BLK,), lambda i: (i,)),
            pl.BlockSpec((_PROJ_BLK,), lambda i: (i,)),
        ],
        out_shape=[
            jax.ShapeDtypeStruct((V,), jnp.float32),
            jax.ShapeDtypeStruct((V,), jnp.float32),
        ],
    )(table_t, W)


def _sc_gather(p0, p1, idx2, L, bpw):
    lpad = ((L + _LANES - 1) // _LANES) * _LANES
    nsl = lpad // _LANES
    gw = _GR * L
    nch = gw // _CH
    assert gw % _CH == 0

    mesh = plsc.VectorSubcoreMesh(core_axis_name="c", subcore_axis_name="s")

    @functools.partial(
        pl.kernel,
        mesh=mesh,
        out_type=jax.ShapeDtypeStruct((_NW, 2, lpad), jnp.float32),
        scratch_types=[
            pltpu.VMEM((bpw * L // 2,), jnp.int32),
            pltpu.VMEM((gw + 8,), jnp.float32),
            pltpu.VMEM((gw + 8,), jnp.float32),
            pltpu.VMEM((gw + 8,), jnp.float32),
            pltpu.VMEM((gw + 8,), jnp.float32),
            pltpu.VMEM((lpad,), jnp.float32),
            pltpu.VMEM((lpad,), jnp.float32),
            pltpu.SemaphoreType.DMA,
            pltpu.SemaphoreType.DMA,
        ],
    )
    def sc_kernel(p0_hbm, p1_hbm, idx_hbm, out_hbm, idx_vm,
                  b00, b01, b10, b11, acc0, acc1, sem0, sem1):
        wid = lax.axis_index("s") * _NC + lax.axis_index("c")
        hw = bpw * L // 2
        ng = hw // gw

        bufs = ((b00, b01), (b10, b11))
        accs = (acc0, acc1)
        zf = jnp.zeros((_LANES,), jnp.float32)
        for a in accs:
            for k in range(nsl):
                a[pl.ds(k * _LANES, _LANES)] = zf
        for pair in bufs:
            for bb in pair:
                bb[pl.ds(gw - 8, _LANES)] = zf

        def fire(g, par, sem):
            for c, pt in ((0, p0_hbm), (1, p1_hbm)):
                for k in range(nch):
                    sl = pl.ds(g * gw + k * _CH, _CH)
                    pltpu.async_copy(
                        pt.at[idx_vm.at[sl]],
                        bufs[par][c].at[pl.ds(k * _CH, _CH)], sem)

        def drain(par, sem):
            for c in range(2):
                pltpu.make_async_copy(
                    p0_hbm.at[pl.ds(0, gw)],
                    bufs[par][c].at[pl.ds(0, gw)], sem).wait()

        def accumulate(par):
            for r in range(_GR):
                for k in range(nsl):
                    a_sl = pl.ds(k * _LANES, _LANES)
                    b_sl = pl.ds(r * L + k * _LANES, _LANES)
                    for c in range(2):
                        accs[c][a_sl] = accs[c][a_sl] + bufs[par][c][b_sl]

        def body(i, carry):
            g0 = 2 * i
            fire(g0 + 1, 1, sem1)
            drain(0, sem0)
            accumulate(0)

            @pl.when(g0 + 2 < ng)
            def _():
                fire(g0 + 2, 0, sem0)

            drain(1, sem1)
            accumulate(1)
            return carry

        for h in range(2):
            pltpu.sync_copy(idx_hbm.at[wid, pl.ds(h * hw, hw)], idx_vm)
            fire(0, 0, sem0)
            lax.fori_loop(0, ng // 2, body, 0)

        pltpu.sync_copy(acc0, out_hbm.at[wid, 0])
        pltpu.sync_copy(acc1, out_hbm.at[wid, 1])

    return sc_kernel(p0, p1, idx2)


def _finish_body(part_ref, b_ref, out_ref, *, inv_b, L):
    s = jnp.sum(part_ref[...], axis=0)
    lpad = s.shape[1]
    li = lax.broadcasted_iota(jnp.int32, (L, lpad), 0)
    ji = lax.broadcasted_iota(jnp.int32, (L, lpad), 1)
    sel = (li == ji).astype(jnp.float32)
    o = lax.dot_general(
        sel, s, dimension_numbers=(((1,), (1,)), ((), ())),
        preferred_element_type=jnp.float32)
    out_ref[...] = o * inv_b + b_ref[...]


def _finish(partials, b2d, B, L):
    return pl.pallas_call(
        functools.partial(_finish_body, inv_b=1.0 / B, L=L),
        out_shape=jax.ShapeDtypeStruct((L, 2), jnp.float32),
    )(partials, b2d)


def kernel(input_sentence_indices, table, W, b):
    idx = input_sentence_indices.astype(jnp.int32)
    B, L = idx.shape
    V, D = table.shape
    C = W.shape[0]
    assert C == 2 and B % _NW == 0 and _CH < L <= 2 * _CH

    bpw = B // _NW
    p0, p1 = _project(table.T, W)
    idx2 = idx.reshape(_NW, bpw * L)
    partials = _sc_gather(p0, p1, idx2, L, bpw)
    return _finish(partials, b.reshape(1, C), B, L)


# speedup vs baseline: 5.8595x; 1.0064x over previous
import functools

import jax
import jax.numpy as jnp
from jax import lax
from jax.experimental import pallas as pl
from jax.experimental.pallas import tpu as pltpu
from jax.experimental.pallas import tpu_sc as plsc

_NC = 2
_NS = 16
_NW = _NC * _NS
_CH = 128
_LANES = 16
_GR = 16

_PROJ_BLK = 65536


def _project_body(table_ref, w_ref, out0_ref, out1_ref):
    r = lax.dot_general(
        w_ref[...], table_ref[...],
        dimension_numbers=(((1,), (0,)), ((), ())),
        preferred_element_type=jnp.float32)
    out0_ref[...] = r[0]
    out1_ref[...] = r[1]


def _project(table_t, W):
    D, V = table_t.shape
    C = W.shape[0]
    return pl.pallas_call(
        _project_body,
        grid=(pl.cdiv(V, _PROJ_BLK),),
        in_specs=[
            pl.BlockSpec((D, _PROJ_BLK), lambda i: (0, i)),
            pl.BlockSpec((C, D), lambda i: (0, 0)),
        ],
        out_specs=[
            pl.BlockSpec((_PROJ_BLK,), lambda i: (i,)),
            pl.BlockSpec((_PROJ_BLK,), lambda i: (i,)),
        ],
        out_shape=[
            jax.ShapeDtypeStruct((V,), jnp.float32),
            jax.ShapeDtypeStruct((V,), jnp.float32),
        ],
    )(table_t, W)


def _sc_gather(p0, p1, idx2, L, bpw):
    lpad = ((L + _LANES - 1) // _LANES) * _LANES
    nsl = lpad // _LANES
    gw = _GR * L
    nch = gw // _CH
    assert gw % _CH == 0

    mesh = plsc.VectorSubcoreMesh(core_axis_name="c", subcore_axis_name="s")

    @functools.partial(
        pl.kernel,
        mesh=mesh,
        out_type=jax.ShapeDtypeStruct((_NW, 2, lpad), jnp.float32),
        scratch_types=[
            pltpu.VMEM((bpw * L // 2,), jnp.int32),
            pltpu.VMEM((gw + 8,), jnp.float32),
            pltpu.VMEM((gw + 8,), jnp.float32),
            pltpu.VMEM((gw + 8,), jnp.float32),
            pltpu.VMEM((gw + 8,), jnp.float32),
            pltpu.VMEM((lpad,), jnp.float32),
            pltpu.VMEM((lpad,), jnp.float32),
            pltpu.SemaphoreType.DMA,
            pltpu.SemaphoreType.DMA,
        ],
    )
    def sc_kernel(p0_hbm, p1_hbm, idx_hbm, out_hbm, idx_vm,
                  b00, b01, b10, b11, acc0, acc1, sem0, sem1):
        wid = lax.axis_index("s") * _NC + lax.axis_index("c")
        hw = bpw * L // 2
        ng = hw // gw

        bufs = ((b00, b01), (b10, b11))
        accs = (acc0, acc1)
        zf = jnp.zeros((_LANES,), jnp.float32)
        for a in accs:
            for k in range(nsl):
                a[pl.ds(k * _LANES, _LANES)] = zf
        for pair in bufs:
            for bb in pair:
                bb[pl.ds(gw - 8, _LANES)] = zf

        def fire(g, par, sem):
            for c, pt in ((0, p0_hbm), (1, p1_hbm)):
                for k in range(nch):
                    sl = pl.ds(g * gw + k * _CH, _CH)
                    pltpu.async_copy(
                        pt.at[idx_vm.at[sl]],
                        bufs[par][c].at[pl.ds(k * _CH, _CH)], sem)

        def drain(par, sem):
            for c in range(2):
                pltpu.make_async_copy(
                    p0_hbm.at[pl.ds(0, gw)],
                    bufs[par][c].at[pl.ds(0, gw)], sem).wait()

        def accumulate(par):
            for r in range(_GR):
                for k in range(nsl):
                    a_sl = pl.ds(k * _LANES, _LANES)
                    b_sl = pl.ds(r * L + k * _LANES, _LANES)
                    for c in range(2):
                        accs[c][a_sl] = accs[c][a_sl] + bufs[par][c][b_sl]

        def body(i, carry):
            g0 = 2 * i
            fire(g0 + 1, 1, sem1)
            drain(0, sem0)
            accumulate(0)

            @pl.when(g0 + 2 < ng)
            def _():
                fire(g0 + 2, 0, sem0)

            drain(1, sem1)
            accumulate(1)
            return carry

        for h in range(2):
            pltpu.sync_copy(idx_hbm.at[wid, pl.ds(h * hw, hw)], idx_vm)
            fire(0, 0, sem0)
            lax.fori_loop(0, ng // 2, body, 0)

        pltpu.sync_copy(acc0, out_hbm.at[wid, 0])
        pltpu.sync_copy(acc1, out_hbm.at[wid, 1])

    return sc_kernel(p0, p1, idx2)


def _finish_body(part_ref, b_ref, out_ref, *, inv_b, L):
    s = jnp.sum(part_ref[...], axis=0)
    lpad = s.shape[1]
    li = lax.broadcasted_iota(jnp.int32, (L, lpad), 0)
    ji = lax.broadcasted_iota(jnp.int32, (L, lpad), 1)
    sel = (li == ji).astype(jnp.float32)
    o = lax.dot_general(
        sel, s, dimension_numbers=(((1,), (1,)), ((), ())),
        preferred_element_type=jnp.float32)
    out_ref[...] = o * inv_b + b_ref[...]


def _finish(partials, b2d, B, L):
    return pl.pallas_call(
        functools.partial(_finish_body, inv_b=1.0 / B, L=L),
        out_shape=jax.ShapeDtypeStruct((L, 2), jnp.float32),
    )(partials, b2d)


def kernel(input_sentence_indices, table, W, b):
    idx = input_sentence_indices.astype(jnp.int32)
    B, L = idx.shape
    V, D = table.shape
    C = W.shape[0]
    assert C == 2 and B % _NW == 0 and _CH < L <= 2 * _CH

    bpw = B // _NW
    p0, p1 = _project(table.T, W)
    idx2 = idx.reshape(_NW, bpw * L)
    partials = _sc_gather(p0, p1, idx2, L, bpw)
    return _finish(partials, b.reshape(1, C), B, L)
